# baseline (speedup 1.0000x reference)
import functools

import jax
import jax.numpy as jnp
from jax import lax
from jax.experimental import pallas as pl
from jax.experimental.pallas import tpu as pltpu
from jax.experimental.pallas import tpu_sc as plsc

_HH = 512
_RECENT = 512
_CACHE = _HH + _RECENT

_B, _H, _Q, _T, _D = 8, 32, 8, 4096, 64
_SEL = _T - _RECENT
_NC, _NS = 2, 16
_NW = _NC * _NS
_PAIRS = _B * _H
_PPW = _PAIRS // _NW
_NCHUNK = _CACHE // 128


def _tc_body(scores_ref, hh_ref, tau_ref, m_ref):
    s = scores_ref[0]
    hh = jnp.sum(s, axis=1)
    hh_ref[0] = hh
    bits = lax.bitcast_convert_type(hh[:, :_SEL], jnp.int32)

    def step(_, carry):
        lo, hi = carry
        mid = lo + lax.div(hi - lo, 2)
        cnt = jnp.sum((bits > mid).astype(jnp.int32), axis=1, keepdims=True)
        conv = cnt < _HH
        return jnp.where(conv, lo, mid + 1), jnp.where(conv, mid, hi)

    lo0 = jnp.zeros((_H, 1), jnp.int32)
    hi0 = jnp.full((_H, 1), jnp.int32(0x7F000000))
    tau, _ = lax.fori_loop(0, 31, step, (lo0, hi0))
    c = jnp.sum((bits > tau).astype(jnp.int32), axis=1, keepdims=True)
    tau_ref[0] = jnp.broadcast_to(tau, (_H, 128))
    m_ref[0] = jnp.broadcast_to(_HH - c, (_H, 128))


def _sc_body(hh_hbm, tau_hbm, m_hbm, k_hbm, v_hbm, kout, vout, hhout,
             hh_v, tau_v, m_v, idx_v, gidx_v, col_v, gbufs, crows, hho_v,
             gsems, wsems):
    wid = lax.axis_index("s") * _NC + lax.axis_index("c")
    pltpu.sync_copy(tau_hbm.at[pl.ds(wid * _PPW, 16)], tau_v)
    pltpu.sync_copy(m_hbm.at[pl.ds(wid * _PPW, 16)], m_v)
    lanes = lax.broadcasted_iota(jnp.int32, (16,), 0)

    def do_pair(p, _):
        pair = wid * _PPW + p
        pltpu.sync_copy(hh_hbm.at[pair], hh_v)
        pidx = jnp.full((16,), p, jnp.int32)
        tau_b = plsc.load_gather(tau_v, [pidx])
        m_b = plsc.load_gather(m_v, [pidx])

        def step(i, carry):
            e, off = carry
            v = hh_v[pl.ds(i * 16, 16)]
            pos = i * 16 + lanes
            gt = v > tau_b
            eq = v == tau_b
            eqc = plsc.cumsum(eq.astype(jnp.int32))
            keep_eq = jnp.logical_and(eq, (e + eqc) <= m_b)
            msk = jnp.logical_or(gt, keep_eq)
            dest = off + plsc.cumsum(msk.astype(jnp.int32)) - 1
            plsc.store_scatter(idx_v, [dest], pos, mask=msk)
            return (e + plsc.all_reduce_population_count(eq),
                    off + plsc.all_reduce_population_count(msk))

        z = jnp.zeros((16,), jnp.int32)
        lax.fori_loop(0, _SEL // 16, step, (z, z))

        def recent(j, _):
            idx_v[pl.ds(_HH + j * 16, 16)] = _SEL + j * 16 + lanes
            return 0

        lax.fori_loop(0, _RECENT // 16, recent, 0)

        base = pair * (_T // 2)

        def gat(j, _):
            iv = idx_v[pl.ds(j * 16, 16)]
            hho_v[pl.ds(j * 16, 16)] = plsc.load_gather(hh_v, [iv])
            gidx_v[j // 8, pl.ds((j % 8) * 16, 16)] = base + \
                lax.shift_right_logical(iv, 1)
            col_v[pl.ds(j * 16, 16)] = jnp.bitwise_and(iv, 1) * _D
            return 0

        lax.fori_loop(0, _CACHE // 16, gat, 0)
        pltpu.sync_copy(hho_v, hhout.at[pair])

        def extract(c, gbuf):
            def erow(j, _):
                zero16 = jnp.zeros((16,), jnp.int32)
                cb = plsc.load_gather(col_v, [c * 128 + j + zero16])
                src = cb + lanes
                row = j + zero16
                for k in range(4):
                    vals = plsc.load_gather(gbuf, [row, src + k * 16])
                    crows[c % 2][pl.ds(j * _D + k * 16, 16)] = vals
                return 0
            lax.fori_loop(0, 128, erow, 0)

        for t, (src_hbm, dst_hbm) in enumerate(((k_hbm, kout), (v_hbm, vout))):
            gathers = [None, None]
            writes = [None, None]
            for c in range(_NCHUNK):
                if c == 0:
                    gathers[0] = pltpu.async_copy(
                        src_hbm.at[gidx_v.at[0]], gbufs[0], gsems[0])
                if c + 1 < _NCHUNK:
                    gathers[(c + 1) % 2] = pltpu.async_copy(
                        src_hbm.at[gidx_v.at[c + 1]], gbufs[(c + 1) % 2],
                        gsems[(c + 1) % 2])
                gathers[c % 2].wait()
                if writes[c % 2] is not None:
                    writes[c % 2].wait()
                extract(c, gbufs[c % 2])
                writes[c % 2] = pltpu.async_copy(
                    crows[c % 2], dst_hbm.at[pair, pl.ds(c * 128 * _D,
                                                         128 * _D)],
                    wsems[c % 2])
            writes[0].wait()
            writes[1].wait()
        return 0

    lax.fori_loop(0, _PPW, do_pair, 0)


@functools.cache
def _make_sc_kernel():
    mesh = plsc.VectorSubcoreMesh(core_axis_name="c", subcore_axis_name="s",
                                  num_cores=_NC, num_subcores=_NS)
    return pl.kernel(
        _sc_body,
        out_type=(
            jax.ShapeDtypeStruct((_PAIRS, _CACHE * _D), jnp.float32),
            jax.ShapeDtypeStruct((_PAIRS, _CACHE * _D), jnp.float32),
            jax.ShapeDtypeStruct((_PAIRS, _CACHE), jnp.float32),
        ),
        mesh=mesh,
        scratch_types=[
            pltpu.VMEM((_T,), jnp.float32),
            pltpu.VMEM((16,), jnp.float32),
            pltpu.VMEM((16,), jnp.int32),
            pltpu.VMEM((_CACHE,), jnp.int32),
            pltpu.VMEM((_NCHUNK, 128), jnp.int32),
            pltpu.VMEM((_CACHE,), jnp.int32),
            [pltpu.VMEM((128, 128), jnp.float32)] * 2,
            [pltpu.VMEM((128 * _D,), jnp.float32)] * 2,
            pltpu.VMEM((_CACHE,), jnp.float32),
            [pltpu.SemaphoreType.DMA] * 2,
            [pltpu.SemaphoreType.DMA] * 2,
        ],
        compiler_params=pltpu.CompilerParams(needs_layout_passes=False),
    )


def kernel(attn_score_cache, key_cache, value_cache):
    hh, tau_bits, m = pl.pallas_call(
        _tc_body,
        grid=(_B,),
        in_specs=[pl.BlockSpec((1, _H, _Q, _T), lambda b: (b, 0, 0, 0))],
        out_specs=[
            pl.BlockSpec((1, _H, _T), lambda b: (b, 0, 0)),
            pl.BlockSpec((1, _H, 128), lambda b: (b, 0, 0)),
            pl.BlockSpec((1, _H, 128), lambda b: (b, 0, 0)),
        ],
        out_shape=[
            jax.ShapeDtypeStruct((_B, _H, _T), jnp.float32),
            jax.ShapeDtypeStruct((_B, _H, 128), jnp.int32),
            jax.ShapeDtypeStruct((_B, _H, 128), jnp.int32),
        ],
    )(attn_score_cache)

    tau = lax.bitcast_convert_type(tau_bits[:, :, 0], jnp.float32).reshape(-1)
    mm = m[:, :, 0].reshape(-1)
    tau_pad = jnp.pad(tau, (0, 16))
    m_pad = jnp.pad(mm, (0, 16))
    kout, vout, hhout = _make_sc_kernel()(
        hh.reshape(_PAIRS, _T), tau_pad, m_pad,
        key_cache.reshape(_PAIRS * _T // 2, 2 * _D),
        value_cache.reshape(_PAIRS * _T // 2, 2 * _D))
    return (kout.reshape(_B, _H, _CACHE, _D),
            vout.reshape(_B, _H, _CACHE, _D),
            hhout.reshape(_B, _H, _CACHE))


# speedup vs baseline: 2.8333x; 2.8333x over previous
import functools

import jax
import jax.numpy as jnp
from jax import lax
from jax.experimental import pallas as pl
from jax.experimental.pallas import tpu as pltpu
from jax.experimental.pallas import tpu_sc as plsc

_HH = 512
_RECENT = 512
_CACHE = _HH + _RECENT

_B, _H, _Q, _T, _D = 8, 32, 8, 4096, 64
_SEL = _T - _RECENT
_NC, _NS = 2, 16
_NW = _NC * _NS
_PAIRS = _B * _H
_PPW = _PAIRS // _NW
_DSLAB = 8
_NSLAB = _D // _DSLAB


def _tc_body(scores_ref, hh_ref, tau_ref, m_ref):
    s = scores_ref[0]
    hh = jnp.sum(s, axis=1)
    hh_ref[0] = hh
    bits = lax.bitcast_convert_type(hh[:, :_SEL], jnp.int32)

    def step(_, carry):
        lo, hi = carry
        mid = lo + lax.div(hi - lo, 2)
        cnt = jnp.sum((bits > mid).astype(jnp.int32), axis=1, keepdims=True)
        conv = cnt < _HH
        return jnp.where(conv, lo, mid + 1), jnp.where(conv, mid, hi)

    lo0 = jnp.zeros((_H, 1), jnp.int32)
    hi0 = jnp.full((_H, 1), jnp.int32(0x7F000000))
    tau, _ = lax.fori_loop(0, 31, step, (lo0, hi0))
    c = jnp.sum((bits > tau).astype(jnp.int32), axis=1, keepdims=True)
    tau_ref[0] = jnp.broadcast_to(tau, (_H, 128))
    m_ref[0] = jnp.broadcast_to(_HH - c, (_H, 128))


def _sc_body(hh_hbm, tau_hbm, m_hbm, k_hbm, v_hbm, kout, vout, hhout,
             hh_v, tau_v, m_v, idx_v, slabs, orows, hho_v, gsems, wsems):
    wid = lax.axis_index("s") * _NC + lax.axis_index("c")
    pltpu.sync_copy(tau_hbm.at[pl.ds(wid * _PPW, 16)], tau_v)
    pltpu.sync_copy(m_hbm.at[pl.ds(wid * _PPW, 16)], m_v)
    lanes = lax.broadcasted_iota(jnp.int32, (16,), 0)

    def do_pair(p, _):
        pair = wid * _PPW + p
        pltpu.sync_copy(hh_hbm.at[pair], hh_v)
        pidx = jnp.full((16,), p, jnp.int32)
        tau_b = plsc.load_gather(tau_v, [pidx])
        m_b = plsc.load_gather(m_v, [pidx])

        def step(i, carry):
            e, off = carry
            v = hh_v[pl.ds(i * 16, 16)]
            pos = i * 16 + lanes
            gt = v > tau_b
            eq = v == tau_b
            eqc = plsc.cumsum(eq.astype(jnp.int32))
            keep_eq = jnp.logical_and(eq, (e + eqc) <= m_b)
            msk = jnp.logical_or(gt, keep_eq)
            dest = off + plsc.cumsum(msk.astype(jnp.int32)) - 1
            plsc.store_scatter(idx_v, [dest], pos, mask=msk)
            return (e + plsc.all_reduce_population_count(eq),
                    off + plsc.all_reduce_population_count(msk))

        z = jnp.zeros((16,), jnp.int32)
        lax.fori_loop(0, _SEL // 16, step, (z, z))

        def recent(j, _):
            idx_v[pl.ds(_HH + j * 16, 16)] = _SEL + j * 16 + lanes
            return 0

        lax.fori_loop(0, _RECENT // 16, recent, 0)

        def hh_gather(j, _):
            iv = idx_v[pl.ds(j * 16, 16)]
            hho_v[pl.ds(j * 16, 16)] = plsc.load_gather(hh_v, [iv])
            return 0

        lax.fori_loop(0, _CACHE // 16, hh_gather, 0)
        pltpu.sync_copy(hho_v, hhout.at[pair])

        def extract(slab, obuf):
            def erow(j, _):
                iv = idx_v[pl.ds(j * 16, 16)]
                for d in range(_DSLAB):
                    dvec = jnp.full((16,), d, jnp.int32)
                    obuf[d, pl.ds(j * 16, 16)] = plsc.load_gather(
                        slab, [dvec, iv])
                return 0
            lax.fori_loop(0, _CACHE // 16, erow, 0)

        rbase = pair * _D
        for src_hbm, dst_hbm in ((k_hbm, kout), (v_hbm, vout)):
            gathers = [None, None]
            writes = [None, None]
            for s in range(_NSLAB):
                if s == 0:
                    gathers[0] = pltpu.async_copy(
                        src_hbm.at[pl.ds(rbase, _DSLAB), :], slabs[0],
                        gsems[0])
                if s + 1 < _NSLAB:
                    gathers[(s + 1) % 2] = pltpu.async_copy(
                        src_hbm.at[pl.ds(rbase + (s + 1) * _DSLAB, _DSLAB), :],
                        slabs[(s + 1) % 2], gsems[(s + 1) % 2])
                gathers[s % 2].wait()
                if writes[s % 2] is not None:
                    writes[s % 2].wait()
                extract(slabs[s % 2], orows[s % 2])
                writes[s % 2] = pltpu.async_copy(
                    orows[s % 2],
                    dst_hbm.at[pl.ds(rbase + s * _DSLAB, _DSLAB), :],
                    wsems[s % 2])
            writes[0].wait()
            writes[1].wait()
        return 0

    lax.fori_loop(0, _PPW, do_pair, 0)


@functools.cache
def _make_sc_kernel():
    mesh = plsc.VectorSubcoreMesh(core_axis_name="c", subcore_axis_name="s",
                                  num_cores=_NC, num_subcores=_NS)
    return pl.kernel(
        _sc_body,
        out_type=(
            jax.ShapeDtypeStruct((_PAIRS * _D, _CACHE), jnp.float32),
            jax.ShapeDtypeStruct((_PAIRS * _D, _CACHE), jnp.float32),
            jax.ShapeDtypeStruct((_PAIRS, _CACHE), jnp.float32),
        ),
        mesh=mesh,
        scratch_types=[
            pltpu.VMEM((_T,), jnp.float32),
            pltpu.VMEM((16,), jnp.float32),
            pltpu.VMEM((16,), jnp.int32),
            pltpu.VMEM((_CACHE,), jnp.int32),
            [pltpu.VMEM((_DSLAB, _T), jnp.float32)] * 2,
            [pltpu.VMEM((_DSLAB, _CACHE), jnp.float32)] * 2,
            pltpu.VMEM((_CACHE,), jnp.float32),
            [pltpu.SemaphoreType.DMA] * 2,
            [pltpu.SemaphoreType.DMA] * 2,
        ],
        compiler_params=pltpu.CompilerParams(needs_layout_passes=False),
    )


def kernel(attn_score_cache, key_cache, value_cache):
    hh, tau_bits, m = pl.pallas_call(
        _tc_body,
        grid=(_B,),
        in_specs=[pl.BlockSpec((1, _H, _Q, _T), lambda b: (b, 0, 0, 0))],
        out_specs=[
            pl.BlockSpec((1, _H, _T), lambda b: (b, 0, 0)),
            pl.BlockSpec((1, _H, 128), lambda b: (b, 0, 0)),
            pl.BlockSpec((1, _H, 128), lambda b: (b, 0, 0)),
        ],
        out_shape=[
            jax.ShapeDtypeStruct((_B, _H, _T), jnp.float32),
            jax.ShapeDtypeStruct((_B, _H, 128), jnp.int32),
            jax.ShapeDtypeStruct((_B, _H, 128), jnp.int32),
        ],
    )(attn_score_cache)

    tau = lax.bitcast_convert_type(tau_bits[:, :, 0], jnp.float32).reshape(-1)
    mm = m[:, :, 0].reshape(-1)
    tau_pad = jnp.pad(tau, (0, 16))
    m_pad = jnp.pad(mm, (0, 16))
    kt = key_cache.transpose(0, 1, 3, 2).reshape(_PAIRS * _D, _T)
    vt = value_cache.transpose(0, 1, 3, 2).reshape(_PAIRS * _D, _T)
    kout_t, vout_t, hhout = _make_sc_kernel()(
        hh.reshape(_PAIRS, _T), tau_pad, m_pad, kt, vt)
    kout = kout_t.reshape(_B, _H, _D, _CACHE).transpose(0, 1, 3, 2)
    vout = vout_t.reshape(_B, _H, _D, _CACHE).transpose(0, 1, 3, 2)
    return kout, vout, hhout.reshape(_B, _H, _CACHE)


# speedup vs baseline: 3.3050x; 1.1665x over previous
import functools

import jax
import jax.numpy as jnp
from jax import lax
from jax.experimental import pallas as pl
from jax.experimental.pallas import tpu as pltpu
from jax.experimental.pallas import tpu_sc as plsc

_HH = 512
_RECENT = 512
_CACHE = _HH + _RECENT

_B, _H, _Q, _T, _D = 8, 32, 8, 4096, 64
_SEL = _T - _RECENT
_NC, _NS = 2, 16
_NW = _NC * _NS
_PAIRS = _B * _H
_PPW = _PAIRS // _NW
_DSLAB = 8
_NSLAB = _D // _DSLAB
_RROWS = 32


def _tc_body(scores_ref, hh_ref, tau_ref, m_ref):
    s = scores_ref[0]
    hh = jnp.sum(s, axis=1)
    hh_ref[0] = hh
    bits = lax.bitcast_convert_type(hh[:, :_SEL], jnp.int32)

    def step(_, carry):
        lo, hi = carry
        mid = lo + lax.div(hi - lo, 2)
        cnt = jnp.sum((bits > mid).astype(jnp.int32), axis=1, keepdims=True)
        conv = cnt < _HH
        return jnp.where(conv, lo, mid + 1), jnp.where(conv, mid, hi)

    lo0 = jnp.zeros((_H, 1), jnp.int32)
    hi0 = jnp.full((_H, 1), jnp.int32(0x7F000000))
    tau, _ = lax.fori_loop(0, 31, step, (lo0, hi0))
    c = jnp.sum((bits > tau).astype(jnp.int32), axis=1, keepdims=True)
    tau_ref[0] = jnp.broadcast_to(tau, (_H, 128))
    m_ref[0] = jnp.broadcast_to(_HH - c, (_H, 128))


def _sc_body(hh_hbm, tau_hbm, m_hbm, k_hbm, v_hbm, kout, vout, hhout,
             hh_v, tau_v, m_v, idx_v, slabs, orows, rbufs, hho_v,
             gsems, wsems, rsems):
    wid = lax.axis_index("s") * _NC + lax.axis_index("c")
    pltpu.sync_copy(tau_hbm.at[pl.ds(wid * _PPW, 16)], tau_v)
    pltpu.sync_copy(m_hbm.at[pl.ds(wid * _PPW, 16)], m_v)
    lanes = lax.broadcasted_iota(jnp.int32, (16,), 0)
    dvecs = [jnp.full((16,), d, jnp.int32) for d in range(_DSLAB)]

    def do_pair(p, _):
        pair = wid * _PPW + p
        pltpu.sync_copy(hh_hbm.at[pair], hh_v)
        pidx = jnp.full((16,), p, jnp.int32)
        tau_b = plsc.load_gather(tau_v, [pidx])
        m_b = plsc.load_gather(m_v, [pidx])

        def step(i, carry):
            e, off = carry
            v = hh_v[pl.ds(i * 16, 16)]
            pos = i * 16 + lanes
            gt = v > tau_b
            eq = v == tau_b
            eqc = plsc.cumsum(eq.astype(jnp.int32))
            keep_eq = jnp.logical_and(eq, (e + eqc) <= m_b)
            msk = jnp.logical_or(gt, keep_eq)
            dest = off + plsc.cumsum(msk.astype(jnp.int32)) - 1
            plsc.store_scatter(idx_v, [dest], pos, mask=msk)
            return (e + plsc.all_reduce_population_count(eq),
                    off + plsc.all_reduce_population_count(msk))

        z = jnp.zeros((16,), jnp.int32)
        lax.fori_loop(0, _SEL // 16, step, (z, z))

        def hh_gather(j, _):
            iv = idx_v[pl.ds(j * 16, 16)]
            hho_v[pl.ds(j * 16, 16)] = plsc.load_gather(hh_v, [iv])
            return 0

        lax.fori_loop(0, _HH // 16, hh_gather, 0)

        def hh_recent(j, _):
            hho_v[pl.ds(_HH + j * 16, 16)] = hh_v[pl.ds(_SEL + j * 16, 16)]
            return 0

        lax.fori_loop(0, _RECENT // 16, hh_recent, 0)
        pltpu.sync_copy(hho_v, hhout.at[pair])

        def extract(slab, obuf):
            def erow(j, _):
                iv = idx_v[pl.ds(j * 16, 16)]
                for d in range(_DSLAB):
                    obuf[d, pl.ds(j * 16, 16)] = plsc.load_gather(
                        slab, [dvecs[d], iv])
                return 0
            lax.fori_loop(0, _HH // 16, erow, 0)

        rbase = pair * _D
        for src_hbm, dst_hbm in ((k_hbm, kout), (v_hbm, vout)):
            rreads = []
            for r in range(_D // _RROWS):
                rreads.append(pltpu.async_copy(
                    src_hbm.at[pl.ds(rbase + r * _RROWS, _RROWS),
                               pl.ds(_SEL, _RECENT)],
                    rbufs[r], rsems[r]))
            gathers = [None, None]
            writes = [None, None]
            for s in range(_NSLAB):
                if s == 0:
                    gathers[0] = pltpu.async_copy(
                        src_hbm.at[pl.ds(rbase, _DSLAB), pl.ds(0, _SEL)],
                        slabs[0], gsems[0])
                if s + 1 < _NSLAB:
                    gathers[(s + 1) % 2] = pltpu.async_copy(
                        src_hbm.at[pl.ds(rbase + (s + 1) * _DSLAB, _DSLAB),
                                   pl.ds(0, _SEL)],
                        slabs[(s + 1) % 2], gsems[(s + 1) % 2])
                gathers[s % 2].wait()
                if writes[s % 2] is not None:
                    writes[s % 2].wait()
                extract(slabs[s % 2], orows[s % 2])
                writes[s % 2] = pltpu.async_copy(
                    orows[s % 2],
                    dst_hbm.at[pl.ds(rbase + s * _DSLAB, _DSLAB),
                               pl.ds(0, _HH)],
                    wsems[s % 2])
            rwrites = []
            for r in range(_D // _RROWS):
                rreads[r].wait()
                rwrites.append(pltpu.async_copy(
                    rbufs[r],
                    dst_hbm.at[pl.ds(rbase + r * _RROWS, _RROWS),
                               pl.ds(_HH, _RECENT)],
                    rsems[r]))
            writes[0].wait()
            writes[1].wait()
            for w in rwrites:
                w.wait()
        return 0

    lax.fori_loop(0, _PPW, do_pair, 0)


@functools.cache
def _make_sc_kernel():
    mesh = plsc.VectorSubcoreMesh(core_axis_name="c", subcore_axis_name="s",
                                  num_cores=_NC, num_subcores=_NS)
    return pl.kernel(
        _sc_body,
        out_type=(
            jax.ShapeDtypeStruct((_PAIRS * _D, _CACHE), jnp.float32),
            jax.ShapeDtypeStruct((_PAIRS * _D, _CACHE), jnp.float32),
            jax.ShapeDtypeStruct((_PAIRS, _CACHE), jnp.float32),
        ),
        mesh=mesh,
        scratch_types=[
            pltpu.VMEM((_T,), jnp.float32),
            pltpu.VMEM((16,), jnp.float32),
            pltpu.VMEM((16,), jnp.int32),
            pltpu.VMEM((_CACHE,), jnp.int32),
            [pltpu.VMEM((_DSLAB, _SEL), jnp.float32)] * 2,
            [pltpu.VMEM((_DSLAB, _HH), jnp.float32)] * 2,
            [pltpu.VMEM((_RROWS, _RECENT), jnp.float32)] * 2,
            pltpu.VMEM((_CACHE,), jnp.float32),
            [pltpu.SemaphoreType.DMA] * 2,
            [pltpu.SemaphoreType.DMA] * 2,
            [pltpu.SemaphoreType.DMA] * 2,
        ],
        compiler_params=pltpu.CompilerParams(needs_layout_passes=False),
    )


def kernel(attn_score_cache, key_cache, value_cache):
    hh, tau_bits, m = pl.pallas_call(
        _tc_body,
        grid=(_B,),
        in_specs=[pl.BlockSpec((1, _H, _Q, _T), lambda b: (b, 0, 0, 0))],
        out_specs=[
            pl.BlockSpec((1, _H, _T), lambda b: (b, 0, 0)),
            pl.BlockSpec((1, _H, 128), lambda b: (b, 0, 0)),
            pl.BlockSpec((1, _H, 128), lambda b: (b, 0, 0)),
        ],
        out_shape=[
            jax.ShapeDtypeStruct((_B, _H, _T), jnp.float32),
            jax.ShapeDtypeStruct((_B, _H, 128), jnp.int32),
            jax.ShapeDtypeStruct((_B, _H, 128), jnp.int32),
        ],
    )(attn_score_cache)

    tau = lax.bitcast_convert_type(tau_bits[:, :, 0], jnp.float32).reshape(-1)
    mm = m[:, :, 0].reshape(-1)
    tau_pad = jnp.pad(tau, (0, 16))
    m_pad = jnp.pad(mm, (0, 16))
    kt = key_cache.transpose(0, 1, 3, 2).reshape(_PAIRS * _D, _T)
    vt = value_cache.transpose(0, 1, 3, 2).reshape(_PAIRS * _D, _T)
    kout_t, vout_t, hhout = _make_sc_kernel()(
        hh.reshape(_PAIRS, _T), tau_pad, m_pad, kt, vt)
    kout = kout_t.reshape(_B, _H, _D, _CACHE).transpose(0, 1, 3, 2)
    vout = vout_t.reshape(_B, _H, _D, _CACHE).transpose(0, 1, 3, 2)
    return kout, vout, hhout.reshape(_B, _H, _CACHE)


# speedup vs baseline: 3.4347x; 1.0392x over previous
import functools

import jax
import jax.numpy as jnp
from jax import lax
from jax.experimental import pallas as pl
from jax.experimental.pallas import tpu as pltpu
from jax.experimental.pallas import tpu_sc as plsc

_HH = 512
_RECENT = 512
_CACHE = _HH + _RECENT

_B, _H, _Q, _T, _D = 8, 32, 8, 4096, 64
_SEL = _T - _RECENT
_NC, _NS = 2, 16
_NW = _NC * _NS
_PAIRS = _B * _H
_PPW = _PAIRS // _NW
_DSLAB = 8
_NSLAB = _D // _DSLAB
_RROWS = 16
_SHALF = _T // 4


def _sc_body(sc_hbm, k_hbm, v_hbm, kout, vout, hhout,
             hh_v, sbuf, hist, idx_v, slabs, orows, rbufs, hho_v,
             gsems, wsems, rsems, ssem):
    wid = lax.axis_index("s") * _NC + lax.axis_index("c")
    lanes = lax.broadcasted_iota(jnp.int32, (16,), 0)
    dvecs = [jnp.full((16,), d, jnp.int32) for d in range(_DSLAB)]
    ones16 = jnp.ones((16,), jnp.int32)
    z16 = jnp.zeros((16,), jnp.int32)

    def do_pair(p, _):
        pair = wid * _PPW + p

        for half in range(4):
            pltpu.async_copy(
                sc_hbm.at[pl.ds(pair * _Q, _Q), pl.ds(half * _SHALF, _SHALF)],
                sbuf, ssem).wait()

            def sum_half(i, _h):
                qs = [sbuf[q, pl.ds(i * 16, 16)] for q in range(_Q)]
                while len(qs) > 1:
                    qs = [qs[a] + qs[a + 1] for a in range(0, len(qs), 2)]
                hh_v[pl.ds(half * _SHALF + i * 16, 16)] = qs[0]
                return 0

            lax.fori_loop(0, _SHALF // 16, sum_half, 0)

        prefix = z16
        above = z16

        for s in (24, 16, 8, 0):
            def zero_hist(i, _h):
                hist[i // 16, pl.ds((i % 16) * 16, 16)] = z16
                return 0

            lax.fori_loop(0, 256, zero_hist, 0)

            def fill(i, carry):
                pfx = carry
                b = plsc.bitcast(hh_v[pl.ds(i * 16, 16)], jnp.int32)
                digit = jnp.bitwise_and(
                    lax.shift_right_logical(b, s), 255)
                if s == 24:
                    plsc.addupdate_scatter(hist, [lanes, digit], ones16)
                else:
                    ok = lax.shift_right_logical(b, s + 8) == pfx
                    plsc.addupdate_scatter(hist, [lanes, digit], ones16,
                                           mask=ok)
                return pfx

            lax.fori_loop(0, _SEL // 16, fill, prefix)

            run = above
            found = jnp.zeros((16,), jnp.bool_)
            digit_sel = z16
            above_sel = z16
            for c in range(15, -1, -1):
                cols = pl.ds(c * 16, 16)
                h = hist[0, cols]
                for l in range(1, 16):
                    h = h + hist[l, cols]
                incl = plsc.cumsum(h)
                tot = jnp.max(incl) + z16
                sfx = tot - incl + h
                cond = (run + sfx) >= _HH
                cnt = plsc.all_reduce_population_count(cond)
                has = cnt > 0
                bstar = cnt - 1
                incl_at_b = jnp.sum(jnp.where(lanes == bstar, incl, 0)) + z16
                take = jnp.logical_and(has, jnp.logical_not(found))
                digit_sel = jnp.where(take, c * 16 + bstar, digit_sel)
                above_sel = jnp.where(take, run + tot - incl_at_b, above_sel)
                found = jnp.logical_or(found, has)
                run = run + tot
            prefix = prefix * 256 + digit_sel
            above = above_sel

        tau_b = plsc.bitcast(prefix, jnp.float32)
        m_b = _HH - above

        def step(i, carry):
            e, off = carry
            v = hh_v[pl.ds(i * 16, 16)]
            pos = i * 16 + lanes
            gt = v > tau_b
            eq = v == tau_b
            eqc = plsc.cumsum(eq.astype(jnp.int32))
            keep_eq = jnp.logical_and(eq, (e + eqc) <= m_b)
            msk = jnp.logical_or(gt, keep_eq)
            dest = off + plsc.cumsum(msk.astype(jnp.int32)) - 1
            plsc.store_scatter(idx_v, [dest], pos, mask=msk)
            return (e + plsc.all_reduce_population_count(eq),
                    off + plsc.all_reduce_population_count(msk))

        lax.fori_loop(0, _SEL // 16, step, (z16, z16))

        def hh_gather(j, _h):
            iv = idx_v[pl.ds(j * 16, 16)]
            hho_v[pl.ds(j * 16, 16)] = plsc.load_gather(hh_v, [iv])
            return 0

        lax.fori_loop(0, _HH // 16, hh_gather, 0)

        def hh_recent(j, _h):
            hho_v[pl.ds(_HH + j * 16, 16)] = hh_v[pl.ds(_SEL + j * 16, 16)]
            return 0

        lax.fori_loop(0, _RECENT // 16, hh_recent, 0)
        pltpu.sync_copy(hho_v, hhout.at[pair])

        def extract(slab, obuf):
            def erow(j, _h):
                iv = idx_v[pl.ds(j * 16, 16)]
                for d in range(_DSLAB):
                    obuf[d, pl.ds(j * 16, 16)] = plsc.load_gather(
                        slab, [dvecs[d], iv])
                return 0
            lax.fori_loop(0, _HH // 16, erow, 0)

        rbase = pair * _D
        for src_hbm, dst_hbm in ((k_hbm, kout), (v_hbm, vout)):
            rreads = {}
            for r in range(2):
                rreads[r] = pltpu.async_copy(
                    src_hbm.at[pl.ds(rbase + r * _RROWS, _RROWS),
                               pl.ds(_SEL, _RECENT)],
                    rbufs[r], rsems[r])
            gathers = [None, None]
            writes = [None, None]
            for s in range(_NSLAB):
                if s == 0:
                    gathers[0] = pltpu.async_copy(
                        src_hbm.at[pl.ds(rbase, _DSLAB), pl.ds(0, _SEL)],
                        slabs[0], gsems[0])
                if s + 1 < _NSLAB:
                    gathers[(s + 1) % 2] = pltpu.async_copy(
                        src_hbm.at[pl.ds(rbase + (s + 1) * _DSLAB, _DSLAB),
                                   pl.ds(0, _SEL)],
                        slabs[(s + 1) % 2], gsems[(s + 1) % 2])
                gathers[s % 2].wait()
                if writes[s % 2] is not None:
                    writes[s % 2].wait()
                extract(slabs[s % 2], orows[s % 2])
                writes[s % 2] = pltpu.async_copy(
                    orows[s % 2],
                    dst_hbm.at[pl.ds(rbase + s * _DSLAB, _DSLAB),
                               pl.ds(0, _HH)],
                    wsems[s % 2])
            rwrites = {}
            for r in range(_D // _RROWS):
                rreads[r].wait()
                rwrites[r] = pltpu.async_copy(
                    rbufs[r % 2],
                    dst_hbm.at[pl.ds(rbase + r * _RROWS, _RROWS),
                               pl.ds(_HH, _RECENT)],
                    rsems[r % 2])
                if r + 2 < _D // _RROWS:
                    rwrites[r].wait()
                    rreads[r + 2] = pltpu.async_copy(
                        src_hbm.at[pl.ds(rbase + (r + 2) * _RROWS, _RROWS),
                                   pl.ds(_SEL, _RECENT)],
                        rbufs[r % 2], rsems[r % 2])
            writes[0].wait()
            writes[1].wait()
            rwrites[_D // _RROWS - 2].wait()
            rwrites[_D // _RROWS - 1].wait()
        return 0

    lax.fori_loop(0, _PPW, do_pair, 0)


@functools.cache
def _make_sc_kernel():
    mesh = plsc.VectorSubcoreMesh(core_axis_name="c", subcore_axis_name="s",
                                  num_cores=_NC, num_subcores=_NS)
    return pl.kernel(
        _sc_body,
        out_type=(
            jax.ShapeDtypeStruct((_PAIRS * _D, _CACHE), jnp.float32),
            jax.ShapeDtypeStruct((_PAIRS * _D, _CACHE), jnp.float32),
            jax.ShapeDtypeStruct((_PAIRS, _CACHE), jnp.float32),
        ),
        mesh=mesh,
        scratch_types=[
            pltpu.VMEM((_T,), jnp.float32),
            pltpu.VMEM((_Q, _SHALF), jnp.float32),
            pltpu.VMEM((16, 256), jnp.int32),
            pltpu.VMEM((_CACHE,), jnp.int32),
            [pltpu.VMEM((_DSLAB, _SEL), jnp.float32)] * 2,
            [pltpu.VMEM((_DSLAB, _HH), jnp.float32)] * 2,
            [pltpu.VMEM((_RROWS, _RECENT), jnp.float32)] * 2,
            pltpu.VMEM((_CACHE,), jnp.float32),
            [pltpu.SemaphoreType.DMA] * 2,
            [pltpu.SemaphoreType.DMA] * 2,
            [pltpu.SemaphoreType.DMA] * 2,
            pltpu.SemaphoreType.DMA,
        ],
        compiler_params=pltpu.CompilerParams(needs_layout_passes=False),
    )


def kernel(attn_score_cache, key_cache, value_cache):
    scores2d = attn_score_cache.reshape(_PAIRS * _Q, _T)
    kt = key_cache.transpose(0, 1, 3, 2).reshape(_PAIRS * _D, _T)
    vt = value_cache.transpose(0, 1, 3, 2).reshape(_PAIRS * _D, _T)
    kout_t, vout_t, hhout = _make_sc_kernel()(scores2d, kt, vt)
    kout = kout_t.reshape(_B, _H, _D, _CACHE).transpose(0, 1, 3, 2)
    vout = vout_t.reshape(_B, _H, _D, _CACHE).transpose(0, 1, 3, 2)
    return kout, vout, hhout.reshape(_B, _H, _CACHE)


# speedup vs baseline: 3.7561x; 1.0936x over previous
import functools

import jax
import jax.numpy as jnp
from jax import lax
from jax.experimental import pallas as pl
from jax.experimental.pallas import tpu as pltpu
from jax.experimental.pallas import tpu_sc as plsc

_HH = 512
_RECENT = 512
_CACHE = _HH + _RECENT

_B, _H, _Q, _T, _D = 8, 32, 8, 4096, 64
_SEL = _T - _RECENT
_NC, _NS = 2, 16
_NW = _NC * _NS
_PAIRS = _B * _H
_PPW = _PAIRS // _NW
_DSLAB = 8
_NSLAB = _D // _DSLAB
_RROWS = 16
_SHALF = _T // 4


def _sc_body(sc_hbm, k_hbm, v_hbm, kout, vout, hhout,
             hh_v, sbufs, hist, idx_v, slabs, orows, rbufs, hho_v,
             gsems, wsems, rsems, ssems):
    wid = lax.axis_index("s") * _NC + lax.axis_index("c")
    lanes = lax.broadcasted_iota(jnp.int32, (16,), 0)
    dvecs = [jnp.full((16,), d, jnp.int32) for d in range(_DSLAB)]
    ones16 = jnp.ones((16,), jnp.int32)
    z16 = jnp.zeros((16,), jnp.int32)

    def do_pair(p, _):
        pair = wid * _PPW + p

        sreads = [None, None]
        sreads[0] = pltpu.async_copy(
            sc_hbm.at[pl.ds(pair * _Q, _Q), pl.ds(0, _SHALF)],
            sbufs[0], ssems[0])
        for half in range(4):
            if half + 1 < 4:
                sreads[(half + 1) % 2] = pltpu.async_copy(
                    sc_hbm.at[pl.ds(pair * _Q, _Q),
                              pl.ds((half + 1) * _SHALF, _SHALF)],
                    sbufs[(half + 1) % 2], ssems[(half + 1) % 2])
            sreads[half % 2].wait()
            sbuf = sbufs[half % 2]

            def sum_half(i, _h):
                qs = [sbuf[q, pl.ds(i * 16, 16)] for q in range(_Q)]
                while len(qs) > 1:
                    qs = [qs[a] + qs[a + 1] for a in range(0, len(qs), 2)]
                hh_v[pl.ds(half * _SHALF + i * 16, 16)] = qs[0]
                return 0

            lax.fori_loop(0, _SHALF // 16, sum_half, 0)

        prefix = z16
        above = z16

        for s in (24, 16, 8, 0):
            def zero_hist(i, _h):
                for cc in range(16):
                    hist[i, pl.ds(cc * 16, 16)] = z16
                return 0

            lax.fori_loop(0, 16, zero_hist, 0)

            def fill(i, carry):
                pfx = carry
                b = plsc.bitcast(hh_v[pl.ds(i * 16, 16)], jnp.int32)
                digit = jnp.bitwise_and(
                    lax.shift_right_logical(b, s), 255)
                if s == 24:
                    plsc.addupdate_scatter(hist, [lanes, digit], ones16)
                else:
                    ok = lax.shift_right_logical(b, s + 8) == pfx
                    plsc.addupdate_scatter(hist, [lanes, digit], ones16,
                                           mask=ok)
                return pfx

            lax.fori_loop(0, _SEL // 16, fill, prefix)

            run = above
            found = jnp.zeros((16,), jnp.bool_)
            digit_sel = z16
            above_sel = z16
            for c in range(15, -1, -1):
                cols = pl.ds(c * 16, 16)
                h = hist[0, cols]
                for l in range(1, 16):
                    h = h + hist[l, cols]
                incl = plsc.cumsum(h)
                tot = jnp.max(incl) + z16
                sfx = tot - incl + h
                cond = (run + sfx) >= _HH
                cnt = plsc.all_reduce_population_count(cond)
                has = cnt > 0
                bstar = cnt - 1
                incl_at_b = jnp.sum(jnp.where(lanes == bstar, incl, 0)) + z16
                take = jnp.logical_and(has, jnp.logical_not(found))
                digit_sel = jnp.where(take, c * 16 + bstar, digit_sel)
                above_sel = jnp.where(take, run + tot - incl_at_b, above_sel)
                found = jnp.logical_or(found, has)
                run = run + tot
            prefix = prefix * 256 + digit_sel
            above = above_sel

        tau_b = plsc.bitcast(prefix, jnp.float32)
        m_b = _HH - above

        def step(i, carry):
            e, off = carry
            v = hh_v[pl.ds(i * 16, 16)]
            pos = i * 16 + lanes
            gt = v > tau_b
            eq = v == tau_b
            eqc = plsc.cumsum(eq.astype(jnp.int32))
            keep_eq = jnp.logical_and(eq, (e + eqc) <= m_b)
            msk = jnp.logical_or(gt, keep_eq)
            dest = off + plsc.cumsum(msk.astype(jnp.int32)) - 1
            plsc.store_scatter(idx_v, [dest], pos, mask=msk)
            return (e + plsc.all_reduce_population_count(eq),
                    off + plsc.all_reduce_population_count(msk))

        lax.fori_loop(0, _SEL // 16, step, (z16, z16))

        def hh_gather(j, _h):
            iv = idx_v[pl.ds(j * 16, 16)]
            hho_v[pl.ds(j * 16, 16)] = plsc.load_gather(hh_v, [iv])
            return 0

        lax.fori_loop(0, _HH // 16, hh_gather, 0)

        def hh_recent(j, _h):
            hho_v[pl.ds(_HH + j * 16, 16)] = hh_v[pl.ds(_SEL + j * 16, 16)]
            return 0

        lax.fori_loop(0, _RECENT // 16, hh_recent, 0)
        pltpu.sync_copy(hho_v, hhout.at[pair])

        def extract(slab, obuf):
            def erow(j, _h):
                iv = idx_v[pl.ds(j * 16, 16)]
                for d in range(_DSLAB):
                    obuf[d, pl.ds(j * 16, 16)] = plsc.load_gather(
                        slab, [dvecs[d], iv])
                return 0
            lax.fori_loop(0, _HH // 16, erow, 0)

        rbase = pair * _D
        for src_hbm, dst_hbm in ((k_hbm, kout), (v_hbm, vout)):
            rreads = {}
            for r in range(2):
                rreads[r] = pltpu.async_copy(
                    src_hbm.at[pl.ds(rbase + r * _RROWS, _RROWS),
                               pl.ds(_SEL, _RECENT)],
                    rbufs[r], rsems[r])
            gathers = [None, None]
            writes = [None, None]
            for s in range(_NSLAB):
                if s == 0:
                    gathers[0] = pltpu.async_copy(
                        src_hbm.at[pl.ds(rbase, _DSLAB), pl.ds(0, _SEL)],
                        slabs[0], gsems[0])
                if s + 1 < _NSLAB:
                    gathers[(s + 1) % 2] = pltpu.async_copy(
                        src_hbm.at[pl.ds(rbase + (s + 1) * _DSLAB, _DSLAB),
                                   pl.ds(0, _SEL)],
                        slabs[(s + 1) % 2], gsems[(s + 1) % 2])
                gathers[s % 2].wait()
                if writes[s % 2] is not None:
                    writes[s % 2].wait()
                extract(slabs[s % 2], orows[s % 2])
                writes[s % 2] = pltpu.async_copy(
                    orows[s % 2],
                    dst_hbm.at[pl.ds(rbase + s * _DSLAB, _DSLAB),
                               pl.ds(0, _HH)],
                    wsems[s % 2])
            rwrites = {}
            for r in range(_D // _RROWS):
                rreads[r].wait()
                rwrites[r] = pltpu.async_copy(
                    rbufs[r % 2],
                    dst_hbm.at[pl.ds(rbase + r * _RROWS, _RROWS),
                               pl.ds(_HH, _RECENT)],
                    rsems[r % 2])
                if r + 2 < _D // _RROWS:
                    rwrites[r].wait()
                    rreads[r + 2] = pltpu.async_copy(
                        src_hbm.at[pl.ds(rbase + (r + 2) * _RROWS, _RROWS),
                                   pl.ds(_SEL, _RECENT)],
                        rbufs[r % 2], rsems[r % 2])
            writes[0].wait()
            writes[1].wait()
            rwrites[_D // _RROWS - 2].wait()
            rwrites[_D // _RROWS - 1].wait()
        return 0

    lax.fori_loop(0, _PPW, do_pair, 0)


@functools.cache
def _make_sc_kernel():
    mesh = plsc.VectorSubcoreMesh(core_axis_name="c", subcore_axis_name="s",
                                  num_cores=_NC, num_subcores=_NS)
    return pl.kernel(
        _sc_body,
        out_type=(
            jax.ShapeDtypeStruct((_PAIRS * _D, _CACHE), jnp.float32),
            jax.ShapeDtypeStruct((_PAIRS * _D, _CACHE), jnp.float32),
            jax.ShapeDtypeStruct((_PAIRS, _CACHE), jnp.float32),
        ),
        mesh=mesh,
        scratch_types=[
            pltpu.VMEM((_T,), jnp.float32),
            [pltpu.VMEM((_Q, _SHALF), jnp.float32)] * 2,
            pltpu.VMEM((16, 256), jnp.int32),
            pltpu.VMEM((_CACHE,), jnp.int32),
            [pltpu.VMEM((_DSLAB, _SEL), jnp.float32)] * 2,
            [pltpu.VMEM((_DSLAB, _HH), jnp.float32)] * 2,
            [pltpu.VMEM((_RROWS, _RECENT), jnp.float32)] * 2,
            pltpu.VMEM((_CACHE,), jnp.float32),
            [pltpu.SemaphoreType.DMA] * 2,
            [pltpu.SemaphoreType.DMA] * 2,
            [pltpu.SemaphoreType.DMA] * 2,
            [pltpu.SemaphoreType.DMA] * 2,
        ],
        compiler_params=pltpu.CompilerParams(needs_layout_passes=False),
    )


def kernel(attn_score_cache, key_cache, value_cache):
    scores2d = attn_score_cache.reshape(_PAIRS * _Q, _T)
    kt = key_cache.transpose(0, 1, 3, 2).reshape(_PAIRS * _D, _T)
    vt = value_cache.transpose(0, 1, 3, 2).reshape(_PAIRS * _D, _T)
    kout_t, vout_t, hhout = _make_sc_kernel()(scores2d, kt, vt)
    kout = kout_t.reshape(_B, _H, _D, _CACHE).transpose(0, 1, 3, 2)
    vout = vout_t.reshape(_B, _H, _D, _CACHE).transpose(0, 1, 3, 2)
    return kout, vout, hhout.reshape(_B, _H, _CACHE)


# speedup vs baseline: 3.8625x; 1.0283x over previous
import functools

import jax
import jax.numpy as jnp
from jax import lax
from jax.experimental import pallas as pl
from jax.experimental.pallas import tpu as pltpu
from jax.experimental.pallas import tpu_sc as plsc

_HH = 512
_RECENT = 512
_CACHE = _HH + _RECENT

_B, _H, _Q, _T, _D = 8, 32, 8, 4096, 64
_SEL = _T - _RECENT
_NC, _NS = 2, 16
_NW = _NC * _NS
_PAIRS = _B * _H
_PPW = _PAIRS // _NW
_DSLAB = 8
_NSLAB = _D // _DSLAB
_RROWS = 16
_SHALF = _T // 4


def _sc_body(sc_hbm, k_hbm, v_hbm, kout, vout, hhout,
             hh_v, sbufs, hist, idx_v, slabs, orows, rbufs, hho_v,
             gsems, wsems, rsems, ssems):
    wid = lax.axis_index("s") * _NC + lax.axis_index("c")
    lanes = lax.broadcasted_iota(jnp.int32, (16,), 0)
    dvecs = [jnp.full((16,), d, jnp.int32) for d in range(_DSLAB)]
    ones16 = jnp.ones((16,), jnp.int32)
    z16 = jnp.zeros((16,), jnp.int32)

    def do_pair(p, _):
        pair = wid * _PPW + p
        rbase = pair * _D

        k_slab0 = pltpu.async_copy(
            k_hbm.at[pl.ds(rbase, _DSLAB), pl.ds(0, _SEL)], slabs[0],
            gsems[0])
        k_rreads = {}
        for r in range(2):
            k_rreads[r] = pltpu.async_copy(
                k_hbm.at[pl.ds(rbase + r * _RROWS, _RROWS),
                         pl.ds(_SEL, _RECENT)],
                rbufs[r], rsems[r])

        sreads = [None, None]
        sreads[0] = pltpu.async_copy(
            sc_hbm.at[pl.ds(pair * _Q, _Q), pl.ds(0, _SHALF)],
            sbufs[0], ssems[0])
        for half in range(4):
            if half + 1 < 4:
                sreads[(half + 1) % 2] = pltpu.async_copy(
                    sc_hbm.at[pl.ds(pair * _Q, _Q),
                              pl.ds((half + 1) * _SHALF, _SHALF)],
                    sbufs[(half + 1) % 2], ssems[(half + 1) % 2])
            sreads[half % 2].wait()
            sbuf = sbufs[half % 2]

            def sum_half(i, _h):
                qs = [sbuf[q, pl.ds(i * 16, 16)] for q in range(_Q)]
                while len(qs) > 1:
                    qs = [qs[a] + qs[a + 1] for a in range(0, len(qs), 2)]
                hh_v[pl.ds(half * _SHALF + i * 16, 16)] = qs[0]
                return 0

            lax.fori_loop(0, _SHALF // 16, sum_half, 0)

        prefix = z16
        above = z16

        for s in (24, 16, 8, 0):
            def zero_hist(i, _h):
                for cc in range(16):
                    hist[i, pl.ds(cc * 16, 16)] = z16
                return 0

            lax.fori_loop(0, 16, zero_hist, 0)

            def fill(i, carry):
                pfx = carry
                for u in range(2):
                    b = plsc.bitcast(hh_v[pl.ds(i * 32 + u * 16, 16)],
                                     jnp.int32)
                    digit = jnp.bitwise_and(
                        lax.shift_right_logical(b, s), 255)
                    if s == 24:
                        plsc.addupdate_scatter(hist, [lanes, digit], ones16)
                    else:
                        ok = lax.shift_right_logical(b, s + 8) == pfx
                        plsc.addupdate_scatter(hist, [lanes, digit], ones16,
                                               mask=ok)
                return pfx

            lax.fori_loop(0, _SEL // 32, fill, prefix)

            run = above
            found = jnp.zeros((16,), jnp.bool_)
            digit_sel = z16
            above_sel = z16
            for c in range(15, -1, -1):
                cols = pl.ds(c * 16, 16)
                h = hist[0, cols]
                for l in range(1, 16):
                    h = h + hist[l, cols]
                incl = plsc.cumsum(h)
                tot = jnp.max(incl) + z16
                sfx = tot - incl + h
                cond = (run + sfx) >= _HH
                cnt = plsc.all_reduce_population_count(cond)
                has = cnt > 0
                bstar = cnt - 1
                incl_at_b = jnp.sum(jnp.where(lanes == bstar, incl, 0)) + z16
                take = jnp.logical_and(has, jnp.logical_not(found))
                digit_sel = jnp.where(take, c * 16 + bstar, digit_sel)
                above_sel = jnp.where(take, run + tot - incl_at_b, above_sel)
                found = jnp.logical_or(found, has)
                run = run + tot
            prefix = prefix * 256 + digit_sel
            above = above_sel

        tau_b = plsc.bitcast(prefix, jnp.float32)
        m_b = _HH - above

        def step(i, carry):
            e, off = carry
            for u in range(2):
                v = hh_v[pl.ds(i * 32 + u * 16, 16)]
                pos = i * 32 + u * 16 + lanes
                gt = v > tau_b
                eq = v == tau_b
                eqc = plsc.cumsum(eq.astype(jnp.int32))
                keep_eq = jnp.logical_and(eq, (e + eqc) <= m_b)
                msk = jnp.logical_or(gt, keep_eq)
                dest = off + plsc.cumsum(msk.astype(jnp.int32)) - 1
                plsc.store_scatter(idx_v, [dest], pos, mask=msk)
                e = e + plsc.all_reduce_population_count(eq)
                off = off + plsc.all_reduce_population_count(msk)
            return (e, off)

        lax.fori_loop(0, _SEL // 32, step, (z16, z16))

        def hh_gather(j, _h):
            iv = idx_v[pl.ds(j * 16, 16)]
            hho_v[pl.ds(j * 16, 16)] = plsc.load_gather(hh_v, [iv])
            return 0

        lax.fori_loop(0, _HH // 16, hh_gather, 0)

        def hh_recent(j, _h):
            hho_v[pl.ds(_HH + j * 16, 16)] = hh_v[pl.ds(_SEL + j * 16, 16)]
            return 0

        lax.fori_loop(0, _RECENT // 16, hh_recent, 0)
        pltpu.sync_copy(hho_v, hhout.at[pair])

        def extract(slab, obuf):
            def erow(j, _h):
                iv = idx_v[pl.ds(j * 16, 16)]
                for d in range(_DSLAB):
                    obuf[d, pl.ds(j * 16, 16)] = plsc.load_gather(
                        slab, [dvecs[d], iv])
                return 0
            lax.fori_loop(0, _HH // 16, erow, 0)

        for src_hbm, dst_hbm in ((k_hbm, kout), (v_hbm, vout)):
            if src_hbm is k_hbm:
                rreads = k_rreads
            else:
                rreads = {}
                for r in range(2):
                    rreads[r] = pltpu.async_copy(
                        src_hbm.at[pl.ds(rbase + r * _RROWS, _RROWS),
                                   pl.ds(_SEL, _RECENT)],
                        rbufs[r], rsems[r])
            gathers = [None, None]
            writes = [None, None]
            for s in range(_NSLAB):
                if s == 0:
                    if src_hbm is k_hbm:
                        gathers[0] = k_slab0
                    else:
                        gathers[0] = pltpu.async_copy(
                            src_hbm.at[pl.ds(rbase, _DSLAB), pl.ds(0, _SEL)],
                            slabs[0], gsems[0])
                if s + 1 < _NSLAB:
                    gathers[(s + 1) % 2] = pltpu.async_copy(
                        src_hbm.at[pl.ds(rbase + (s + 1) * _DSLAB, _DSLAB),
                                   pl.ds(0, _SEL)],
                        slabs[(s + 1) % 2], gsems[(s + 1) % 2])
                gathers[s % 2].wait()
                if writes[s % 2] is not None:
                    writes[s % 2].wait()
                extract(slabs[s % 2], orows[s % 2])
                writes[s % 2] = pltpu.async_copy(
                    orows[s % 2],
                    dst_hbm.at[pl.ds(rbase + s * _DSLAB, _DSLAB),
                               pl.ds(0, _HH)],
                    wsems[s % 2])
            rwrites = {}
            for r in range(_D // _RROWS):
                rreads[r].wait()
                rwrites[r] = pltpu.async_copy(
                    rbufs[r % 2],
                    dst_hbm.at[pl.ds(rbase + r * _RROWS, _RROWS),
                               pl.ds(_HH, _RECENT)],
                    rsems[r % 2])
                if r + 2 < _D // _RROWS:
                    rwrites[r].wait()
                    rreads[r + 2] = pltpu.async_copy(
                        src_hbm.at[pl.ds(rbase + (r + 2) * _RROWS, _RROWS),
                                   pl.ds(_SEL, _RECENT)],
                        rbufs[r % 2], rsems[r % 2])
            writes[0].wait()
            writes[1].wait()
            rwrites[_D // _RROWS - 2].wait()
            rwrites[_D // _RROWS - 1].wait()
        return 0

    lax.fori_loop(0, _PPW, do_pair, 0)


@functools.cache
def _make_sc_kernel():
    mesh = plsc.VectorSubcoreMesh(core_axis_name="c", subcore_axis_name="s",
                                  num_cores=_NC, num_subcores=_NS)
    return pl.kernel(
        _sc_body,
        out_type=(
            jax.ShapeDtypeStruct((_PAIRS * _D, _CACHE), jnp.float32),
            jax.ShapeDtypeStruct((_PAIRS * _D, _CACHE), jnp.float32),
            jax.ShapeDtypeStruct((_PAIRS, _CACHE), jnp.float32),
        ),
        mesh=mesh,
        scratch_types=[
            pltpu.VMEM((_T,), jnp.float32),
            [pltpu.VMEM((_Q, _SHALF), jnp.float32)] * 2,
            pltpu.VMEM((16, 256), jnp.int32),
            pltpu.VMEM((_CACHE,), jnp.int32),
            [pltpu.VMEM((_DSLAB, _SEL), jnp.float32)] * 2,
            [pltpu.VMEM((_DSLAB, _HH), jnp.float32)] * 2,
            [pltpu.VMEM((_RROWS, _RECENT), jnp.float32)] * 2,
            pltpu.VMEM((_CACHE,), jnp.float32),
            [pltpu.SemaphoreType.DMA] * 2,
            [pltpu.SemaphoreType.DMA] * 2,
            [pltpu.SemaphoreType.DMA] * 2,
            [pltpu.SemaphoreType.DMA] * 2,
        ],
        compiler_params=pltpu.CompilerParams(needs_layout_passes=False),
    )


def kernel(attn_score_cache, key_cache, value_cache):
    scores2d = attn_score_cache.reshape(_PAIRS * _Q, _T)
    kt = key_cache.transpose(0, 1, 3, 2).reshape(_PAIRS * _D, _T)
    vt = value_cache.transpose(0, 1, 3, 2).reshape(_PAIRS * _D, _T)
    kout_t, vout_t, hhout = _make_sc_kernel()(scores2d, kt, vt)
    kout = kout_t.reshape(_B, _H, _D, _CACHE).transpose(0, 1, 3, 2)
    vout = vout_t.reshape(_B, _H, _D, _CACHE).transpose(0, 1, 3, 2)
    return kout, vout, hhout.reshape(_B, _H, _CACHE)


# speedup vs baseline: 3.9928x; 1.0338x over previous
import functools

import jax
import jax.numpy as jnp
from jax import lax
from jax.experimental import pallas as pl
from jax.experimental.pallas import tpu as pltpu
from jax.experimental.pallas import tpu_sc as plsc

_HH = 512
_RECENT = 512
_CACHE = _HH + _RECENT

_B, _H, _Q, _T, _D = 8, 32, 8, 4096, 64
_SEL = _T - _RECENT
_NC, _NS = 2, 16
_NW = _NC * _NS
_PAIRS = _B * _H
_PPW = _PAIRS // _NW
_DSLAB = 8
_NSLAB = _D // _DSLAB
_RROWS = 16
_SHALF = _T // 4


def _sc_body(sc_hbm, k_hbm, v_hbm, kout, vout, hhout,
             hh_v, sbufs, hist, idx_v, slabs, orows, rbufs, hho_v,
             gsems, wsems, rsems, ssems):
    wid = lax.axis_index("s") * _NC + lax.axis_index("c")
    lanes = lax.broadcasted_iota(jnp.int32, (16,), 0)
    dvecs = [jnp.full((16,), d, jnp.int32) for d in range(_DSLAB)]
    ones16 = jnp.ones((16,), jnp.int32)
    z16 = jnp.zeros((16,), jnp.int32)

    def do_pair(p, _):
        pair = wid * _PPW + p
        rbase = pair * _D

        k_slab0 = pltpu.async_copy(
            k_hbm.at[pl.ds(rbase, _DSLAB), pl.ds(0, _SEL)], slabs[0],
            gsems[0])
        k_rreads = {}
        for r in range(2):
            k_rreads[r] = pltpu.async_copy(
                k_hbm.at[pl.ds(rbase + r * _RROWS, _RROWS),
                         pl.ds(_SEL, _RECENT)],
                rbufs[r], rsems[r])

        sreads = [None, None]
        sreads[0] = pltpu.async_copy(
            sc_hbm.at[pl.ds(pair * _Q, _Q), pl.ds(0, _SHALF)],
            sbufs[0], ssems[0])
        for half in range(4):
            if half + 1 < 4:
                sreads[(half + 1) % 2] = pltpu.async_copy(
                    sc_hbm.at[pl.ds(pair * _Q, _Q),
                              pl.ds((half + 1) * _SHALF, _SHALF)],
                    sbufs[(half + 1) % 2], ssems[(half + 1) % 2])
            sreads[half % 2].wait()
            sbuf = sbufs[half % 2]

            def sum_half(i, _h):
                for u in range(2):
                    qs = [sbuf[q, pl.ds(i * 32 + u * 16, 16)]
                          for q in range(_Q)]
                    while len(qs) > 1:
                        qs = [qs[a] + qs[a + 1] for a in range(0, len(qs), 2)]
                    hh_v[pl.ds(half * _SHALF + i * 32 + u * 16, 16)] = qs[0]
                return 0

            lax.fori_loop(0, _SHALF // 32, sum_half, 0)

        prefix = z16
        above = z16

        for s in (24, 16, 8, 0):
            def zero_hist(i, _h):
                for cc in range(16):
                    hist[i, pl.ds(cc * 16, 16)] = z16
                return 0

            lax.fori_loop(0, 16, zero_hist, 0)

            def fill(i, carry):
                pfx = carry
                for u in range(2):
                    b = plsc.bitcast(hh_v[pl.ds(i * 32 + u * 16, 16)],
                                     jnp.int32)
                    digit = jnp.bitwise_and(
                        lax.shift_right_logical(b, s), 255)
                    if s == 24:
                        plsc.addupdate_scatter(hist, [lanes, digit], ones16)
                    else:
                        ok = lax.shift_right_logical(b, s + 8) == pfx
                        plsc.addupdate_scatter(hist, [lanes, digit], ones16,
                                               mask=ok)
                return pfx

            lax.fori_loop(0, _SEL // 32, fill, prefix)

            run = above
            found = jnp.zeros((16,), jnp.bool_)
            digit_sel = z16
            above_sel = z16
            for c in range(15, -1, -1):
                cols = pl.ds(c * 16, 16)
                h = hist[0, cols]
                for l in range(1, 16):
                    h = h + hist[l, cols]
                incl = plsc.cumsum(h)
                tot = jnp.max(incl) + z16
                sfx = tot - incl + h
                cond = (run + sfx) >= _HH
                cnt = plsc.all_reduce_population_count(cond)
                has = cnt > 0
                bstar = cnt - 1
                incl_at_b = jnp.sum(jnp.where(lanes == bstar, incl, 0)) + z16
                take = jnp.logical_and(has, jnp.logical_not(found))
                digit_sel = jnp.where(take, c * 16 + bstar, digit_sel)
                above_sel = jnp.where(take, run + tot - incl_at_b, above_sel)
                found = jnp.logical_or(found, has)
                run = run + tot
            prefix = prefix * 256 + digit_sel
            above = above_sel

        tau_b = plsc.bitcast(prefix, jnp.float32)
        m_b = _HH - above

        def step(i, carry):
            e, off = carry
            for u in range(2):
                v = hh_v[pl.ds(i * 32 + u * 16, 16)]
                pos = i * 32 + u * 16 + lanes
                gt = v > tau_b
                eq = v == tau_b
                eqc = plsc.cumsum(eq.astype(jnp.int32))
                keep_eq = jnp.logical_and(eq, (e + eqc) <= m_b)
                msk = jnp.logical_or(gt, keep_eq)
                dest = off + plsc.cumsum(msk.astype(jnp.int32)) - 1
                plsc.store_scatter(idx_v, [dest], pos, mask=msk)
                e = e + plsc.all_reduce_population_count(eq)
                off = off + plsc.all_reduce_population_count(msk)
            return (e, off)

        lax.fori_loop(0, _SEL // 32, step, (z16, z16))

        def hh_gather(j, _h):
            iv = idx_v[pl.ds(j * 16, 16)]
            hho_v[pl.ds(j * 16, 16)] = plsc.load_gather(hh_v, [iv])
            return 0

        lax.fori_loop(0, _HH // 16, hh_gather, 0)

        def hh_recent(j, _h):
            hho_v[pl.ds(_HH + j * 16, 16)] = hh_v[pl.ds(_SEL + j * 16, 16)]
            return 0

        lax.fori_loop(0, _RECENT // 16, hh_recent, 0)
        pltpu.sync_copy(hho_v, hhout.at[pair])

        def extract(slab, obuf):
            def erow(j, _h):
                for u in range(2):
                    iv = idx_v[pl.ds(j * 32 + u * 16, 16)]
                    for d in range(_DSLAB):
                        obuf[d, pl.ds(j * 32 + u * 16, 16)] = \
                            plsc.load_gather(slab, [dvecs[d], iv])
                return 0
            lax.fori_loop(0, _HH // 32, erow, 0)

        seq = [(k_hbm, kout, s) for s in range(_NSLAB)] + \
              [(v_hbm, vout, s) for s in range(_NSLAB)]
        gathers = {0: k_slab0}
        writes = {}
        rreads = k_rreads
        rwrites = {}
        for t in range(len(seq)):
            src_hbm, dst_hbm, s = seq[t]
            if t + 1 < len(seq):
                nsrc, _nd, ns = seq[t + 1]
                gathers[t + 1] = pltpu.async_copy(
                    nsrc.at[pl.ds(pair * _D + ns * _DSLAB, _DSLAB),
                            pl.ds(0, _SEL)],
                    slabs[(t + 1) % 2], gsems[(t + 1) % 2])
            gathers[t].wait()
            if t - 2 in writes:
                writes[t - 2].wait()
            extract(slabs[t % 2], orows[t % 2])
            writes[t] = pltpu.async_copy(
                orows[t % 2],
                dst_hbm.at[pl.ds(rbase + s * _DSLAB, _DSLAB),
                           pl.ds(0, _HH)],
                wsems[t % 2])
            if t == _NSLAB:
                for r in range(_D // _RROWS):
                    rreads[r].wait()
                    rwrites[r] = pltpu.async_copy(
                        rbufs[r % 2],
                        kout.at[pl.ds(rbase + r * _RROWS, _RROWS),
                                pl.ds(_HH, _RECENT)],
                        rsems[r % 2])
                    if r + 2 < _D // _RROWS:
                        rwrites[r].wait()
                        rreads[r + 2] = pltpu.async_copy(
                            k_hbm.at[pl.ds(rbase + (r + 2) * _RROWS, _RROWS),
                                     pl.ds(_SEL, _RECENT)],
                            rbufs[r % 2], rsems[r % 2])
                rwrites[_D // _RROWS - 2].wait()
                rwrites[_D // _RROWS - 1].wait()
                rreads = {}
                for r in range(2):
                    rreads[r] = pltpu.async_copy(
                        v_hbm.at[pl.ds(rbase + r * _RROWS, _RROWS),
                                 pl.ds(_SEL, _RECENT)],
                        rbufs[r], rsems[r])
        for r in range(_D // _RROWS):
            rreads[r].wait()
            rwrites[r] = pltpu.async_copy(
                rbufs[r % 2],
                vout.at[pl.ds(rbase + r * _RROWS, _RROWS),
                        pl.ds(_HH, _RECENT)],
                rsems[r % 2])
            if r + 2 < _D // _RROWS:
                rwrites[r].wait()
                rreads[r + 2] = pltpu.async_copy(
                    v_hbm.at[pl.ds(rbase + (r + 2) * _RROWS, _RROWS),
                             pl.ds(_SEL, _RECENT)],
                    rbufs[r % 2], rsems[r % 2])
        rwrites[_D // _RROWS - 2].wait()
        rwrites[_D // _RROWS - 1].wait()
        writes[len(seq) - 2].wait()
        writes[len(seq) - 1].wait()
        return 0

    lax.fori_loop(0, _PPW, do_pair, 0)


@functools.cache
def _make_sc_kernel():
    mesh = plsc.VectorSubcoreMesh(core_axis_name="c", subcore_axis_name="s",
                                  num_cores=_NC, num_subcores=_NS)
    return pl.kernel(
        _sc_body,
        out_type=(
            jax.ShapeDtypeStruct((_PAIRS * _D, _CACHE), jnp.float32),
            jax.ShapeDtypeStruct((_PAIRS * _D, _CACHE), jnp.float32),
            jax.ShapeDtypeStruct((_PAIRS, _CACHE), jnp.float32),
        ),
        mesh=mesh,
        scratch_types=[
            pltpu.VMEM((_T,), jnp.float32),
            [pltpu.VMEM((_Q, _SHALF), jnp.float32)] * 2,
            pltpu.VMEM((16, 256), jnp.int32),
            pltpu.VMEM((_CACHE,), jnp.int32),
            [pltpu.VMEM((_DSLAB, _SEL), jnp.float32)] * 2,
            [pltpu.VMEM((_DSLAB, _HH), jnp.float32)] * 2,
            [pltpu.VMEM((_RROWS, _RECENT), jnp.float32)] * 2,
            pltpu.VMEM((_CACHE,), jnp.float32),
            [pltpu.SemaphoreType.DMA] * 2,
            [pltpu.SemaphoreType.DMA] * 2,
            [pltpu.SemaphoreType.DMA] * 2,
            [pltpu.SemaphoreType.DMA] * 2,
        ],
        compiler_params=pltpu.CompilerParams(needs_layout_passes=False),
    )


def kernel(attn_score_cache, key_cache, value_cache):
    scores2d = attn_score_cache.reshape(_PAIRS * _Q, _T)
    kt = key_cache.transpose(0, 1, 3, 2).reshape(_PAIRS * _D, _T)
    vt = value_cache.transpose(0, 1, 3, 2).reshape(_PAIRS * _D, _T)
    kout_t, vout_t, hhout = _make_sc_kernel()(scores2d, kt, vt)
    kout = kout_t.reshape(_B, _H, _D, _CACHE).transpose(0, 1, 3, 2)
    vout = vout_t.reshape(_B, _H, _D, _CACHE).transpose(0, 1, 3, 2)
    return kout, vout, hhout.reshape(_B, _H, _CACHE)
